# Initial kernel scaffold; baseline (speedup 1.0000x reference)
import jax
import jax.numpy as jnp
from jax.experimental import pallas as pl


def kernel(center_emb, context_emb, center_words, pos_context_words, neg_context_words):
    raise NotImplementedError("write your pallas kernel here")


# speedup vs baseline: 2.2338x; 2.2338x over previous
import functools

import jax
import jax.numpy as jnp
from jax import lax
from jax.experimental import pallas as pl
from jax.experimental.pallas import tpu as pltpu
from jax.experimental.pallas import tpu_sc as plsc

VOCAB = 100000
EMB = 128
BATCH = 16384
NEG = 20

NUM_WORKERS = 32
CHUNK = 128

B_PER_W = BATCH // NUM_WORKERS
N_PER_W = BATCH * NEG // NUM_WORKERS

_mesh = plsc.VectorSubcoreMesh(core_axis_name="c", subcore_axis_name="s")


@functools.partial(
    pl.kernel,
    out_type=(
        jax.ShapeDtypeStruct((BATCH, EMB), jnp.float32),
        jax.ShapeDtypeStruct((BATCH, EMB), jnp.float32),
        jax.ShapeDtypeStruct((BATCH * NEG, EMB), jnp.float32),
    ),
    mesh=_mesh,
    scratch_types=[
        pltpu.VMEM((CHUNK,), jnp.int32),
        pltpu.VMEM((CHUNK, EMB), jnp.float32),
        pltpu.SemaphoreType.DMA,
    ],
)
def _sc_gather(center_hbm, context_hbm, cw_hbm, pw_hbm, nw_hbm,
               v_out, upos_out, uneg_out, idx_v, rows_v, sem):
    wid = lax.axis_index("s") * 2 + lax.axis_index("c")

    def gather_slice(table, idx_hbm, out_hbm, base, nchunks):
        @pl.loop(0, nchunks)
        def _(j):
            off = base + j * CHUNK
            pltpu.sync_copy(idx_hbm.at[pl.ds(off, CHUNK)], idx_v)
            pltpu.async_copy(table.at[idx_v], rows_v, sem).wait()
            pltpu.sync_copy(rows_v, out_hbm.at[pl.ds(off, CHUNK)])

    gather_slice(center_hbm, cw_hbm, v_out, wid * B_PER_W, B_PER_W // CHUNK)
    gather_slice(context_hbm, pw_hbm, upos_out, wid * B_PER_W, B_PER_W // CHUNK)
    gather_slice(context_hbm, nw_hbm, uneg_out, wid * N_PER_W, N_PER_W // CHUNK)


def _log_sigmoid(x):
    return jnp.minimum(x, 0.0) - jnp.log(1.0 + jnp.exp(-jnp.abs(x)))


BB = 512


def _loss_body(v_ref, upos_ref, uneg_ref, out_ref):
    i = pl.program_id(0)
    v = v_ref[...]
    pos = jnp.sum(v * upos_ref[...], axis=1)
    acc = _log_sigmoid(pos)
    for k in range(NEG):
        s = jnp.sum(uneg_ref[:, k, :] * v, axis=1)
        acc += _log_sigmoid(-s)
    block_sum = jnp.sum(acc).reshape(1, 1)

    @pl.when(i == 0)
    def _():
        out_ref[...] = jnp.zeros((1, 1), jnp.float32)

    out_ref[...] += block_sum

    @pl.when(i == pl.num_programs(0) - 1)
    def _():
        out_ref[...] = out_ref[...] * (-1.0 / BATCH)


_loss_call = pl.pallas_call(
    _loss_body,
    grid=(BATCH // BB,),
    in_specs=[
        pl.BlockSpec((BB, EMB), lambda i: (i, 0)),
        pl.BlockSpec((BB, EMB), lambda i: (i, 0)),
        pl.BlockSpec((BB, NEG, EMB), lambda i: (i, 0, 0)),
    ],
    out_specs=pl.BlockSpec((1, 1), lambda i: (0, 0)),
    out_shape=jax.ShapeDtypeStruct((1, 1), jnp.float32),
)


def kernel(center_emb, context_emb, center_words, pos_context_words, neg_context_words):
    cw = center_words.astype(jnp.int32)
    pw = pos_context_words.astype(jnp.int32)
    nw = neg_context_words.astype(jnp.int32).reshape(BATCH * NEG)
    v, u_pos, u_neg = _sc_gather(center_emb, context_emb, cw, pw, nw)
    loss = _loss_call(v, u_pos, u_neg.reshape(BATCH, NEG, EMB))
    return jnp.reshape(loss, ())
